# Initial kernel scaffold; baseline (speedup 1.0000x reference)
import functools

import jax
import jax.numpy as jnp
from jax import lax
from jax.experimental import pallas as pl
from jax.experimental.pallas import tpu as pltpu
from jax.experimental.pallas import tpu_sc as plsc

NC = 2
NS = 16
NW = NC * NS
CHUNK = 2000
ROWS = 2048

_mesh = plsc.VectorSubcoreMesh(core_axis_name="c", subcore_axis_name="s")


def _sc_deg(dst, zeros1, n_pad, epw):
    @functools.partial(
        pl.kernel,
        mesh=_mesh,
        out_type=jax.ShapeDtypeStruct((NC, n_pad), jnp.float32),
        scratch_types=[
            pltpu.VMEM((CHUNK,), jnp.int32),
            pltpu.VMEM((CHUNK,), jnp.float32),
            pltpu.VMEM_SHARED((n_pad,), jnp.float32),
        ],
    )
    def deg_kernel(dst_hbm, zeros_hbm, out_hbm, idx_v, ones_v, acc):
        cid = lax.axis_index("c")
        sid = lax.axis_index("s")
        wid = sid * NC + cid
        slc = n_pad // NS

        @pl.loop(0, CHUNK, step=16)
        def _(i):
            ones_v[pl.ds(i, 16)] = jnp.ones((16,), jnp.float32)

        pltpu.sync_copy(zeros_hbm.at[pl.ds(sid * slc, slc)],
                        acc.at[pl.ds(sid * slc, slc)])
        plsc.subcore_barrier()

        base = wid * epw

        @pl.loop(0, epw, step=CHUNK)
        def _(e0):
            pltpu.sync_copy(dst_hbm.at[pl.ds(base + e0, CHUNK)], idx_v)
            pltpu.sync_copy(ones_v, acc.at[idx_v], add=True)

        plsc.subcore_barrier()
        pltpu.sync_copy(acc.at[pl.ds(sid * slc, slc)],
                        out_hbm.at[cid, pl.ds(sid * slc, slc)])

    return deg_kernel(dst, zeros1)


def _sc_agg(g, src, dst, zeros, n_pad, epw, d):
    @functools.partial(
        pl.kernel,
        mesh=_mesh,
        out_type=jax.ShapeDtypeStruct((NC, n_pad, d), jnp.float32),
        scratch_types=[
            pltpu.VMEM((CHUNK,), jnp.int32),
            pltpu.VMEM((CHUNK,), jnp.int32),
            pltpu.VMEM((CHUNK, d), jnp.float32),
            pltpu.VMEM_SHARED((n_pad, d), jnp.float32),
        ],
    )
    def agg_kernel(g_hbm, src_hbm, dst_hbm, zeros_hbm, out_hbm,
                   sidx, didx, rows, acc):
        cid = lax.axis_index("c")
        sid = lax.axis_index("s")
        wid = sid * NC + cid
        slc = n_pad // NS

        pltpu.sync_copy(zeros_hbm.at[pl.ds(sid * slc, slc)],
                        acc.at[pl.ds(sid * slc, slc)])
        plsc.subcore_barrier()

        base = wid * epw

        @pl.loop(0, epw, step=CHUNK)
        def _(e0):
            pltpu.sync_copy(src_hbm.at[pl.ds(base + e0, CHUNK)], sidx)
            pltpu.sync_copy(dst_hbm.at[pl.ds(base + e0, CHUNK)], didx)
            pltpu.sync_copy(g_hbm.at[sidx], rows)
            pltpu.sync_copy(rows, acc.at[didx], add=True)

        plsc.subcore_barrier()
        pltpu.sync_copy(acc.at[pl.ds(sid * slc, slc)],
                        out_hbm.at[cid, pl.ds(sid * slc, slc)])

    return agg_kernel(g, src, dst, zeros)


def _tc_prep(p0, p1, x_pad, n_pad):
    def body(p0_r, p1_r, x_r, dinv_o, g1_o):
        deg = p0_r[...] + p1_r[...] + 1.0
        dinv = lax.rsqrt(deg)
        dinv_o[...] = dinv
        g1_o[...] = x_r[...] * dinv

    grid = (n_pad // ROWS,)
    return pl.pallas_call(
        body,
        grid=grid,
        in_specs=[
            pl.BlockSpec((ROWS, 1), lambda i: (i, 0)),
            pl.BlockSpec((ROWS, 1), lambda i: (i, 0)),
            pl.BlockSpec((ROWS, 4), lambda i: (i, 0)),
        ],
        out_specs=[
            pl.BlockSpec((ROWS, 1), lambda i: (i, 0)),
            pl.BlockSpec((ROWS, 4), lambda i: (i, 0)),
        ],
        out_shape=[
            jax.ShapeDtypeStruct((n_pad, 1), jnp.float32),
            jax.ShapeDtypeStruct((n_pad, 4), jnp.float32),
        ],
    )(p0, p1, x_pad)


def _tc_layer1(a0, a1, g1, dinv, W1, b1, n_pad):
    def body(a0_r, a1_r, g1_r, dinv_r, w_r, b_r, g2_o):
        s = a0_r[...] + a1_r[...] + g1_r[...]
        h = jnp.dot(s, w_r[...], preferred_element_type=jnp.float32,
                    precision=lax.Precision.HIGHEST)
        h = h * dinv_r[...] + b_r[...]
        g2_o[...] = jnp.maximum(h, 0.0) * dinv_r[...]

    grid = (n_pad // ROWS,)
    return pl.pallas_call(
        body,
        grid=grid,
        in_specs=[
            pl.BlockSpec((ROWS, 4), lambda i: (i, 0)),
            pl.BlockSpec((ROWS, 4), lambda i: (i, 0)),
            pl.BlockSpec((ROWS, 4), lambda i: (i, 0)),
            pl.BlockSpec((ROWS, 1), lambda i: (i, 0)),
            pl.BlockSpec((4, 16), lambda i: (0, 0)),
            pl.BlockSpec((1, 16), lambda i: (0, 0)),
        ],
        out_specs=pl.BlockSpec((ROWS, 16), lambda i: (i, 0)),
        out_shape=jax.ShapeDtypeStruct((n_pad, 16), jnp.float32),
    )(a0, a1, g1, dinv, W1, b1)


def _tc_layer2_pool(q0, q1, g2, dinv, batch_row, W2, b2, Wfc, bfc, n_pad, nb):
    nsteps = n_pad // ROWS

    def body(q0_r, q1_r, g2_r, dinv_r, bt_r, w_r, b_r, wfc_r, bfc_r,
             out_o, sums, counts):
        i = pl.program_id(0)

        @pl.when(i == 0)
        def _():
            sums[...] = jnp.zeros_like(sums)
            counts[...] = jnp.zeros_like(counts)

        s = q0_r[...] + q1_r[...] + g2_r[...]
        h = jnp.dot(s, w_r[...], preferred_element_type=jnp.float32,
                    precision=lax.Precision.HIGHEST)
        h = jnp.maximum(h * dinv_r[...] + b_r[...], 0.0)
        onehot_t = (bt_r[...] ==
                    lax.broadcasted_iota(jnp.int32, (nb, 1), 0)
                    ).astype(jnp.float32)
        sums[...] += jnp.dot(onehot_t, h, preferred_element_type=jnp.float32,
                             precision=lax.Precision.HIGHEST)
        counts[...] += jnp.sum(onehot_t, axis=1, keepdims=True)

        @pl.when(i == nsteps - 1)
        def _():
            pooled = sums[...] / jnp.maximum(counts[...], 1.0)
            z = jnp.dot(pooled, wfc_r[...], preferred_element_type=jnp.float32,
                        precision=lax.Precision.HIGHEST) + bfc_r[...]
            out_o[...] = jax.nn.sigmoid(z)

    return pl.pallas_call(
        body,
        grid=(nsteps,),
        in_specs=[
            pl.BlockSpec((ROWS, 16), lambda i: (i, 0)),
            pl.BlockSpec((ROWS, 16), lambda i: (i, 0)),
            pl.BlockSpec((ROWS, 16), lambda i: (i, 0)),
            pl.BlockSpec((ROWS, 1), lambda i: (i, 0)),
            pl.BlockSpec((1, ROWS), lambda i: (0, i)),
            pl.BlockSpec((16, 32), lambda i: (0, 0)),
            pl.BlockSpec((1, 32), lambda i: (0, 0)),
            pl.BlockSpec((32, 1), lambda i: (0, 0)),
            pl.BlockSpec((1, 1), lambda i: (0, 0)),
        ],
        out_specs=pl.BlockSpec((nb, 1), lambda i: (0, 0)),
        out_shape=jax.ShapeDtypeStruct((nb, 1), jnp.float32),
        scratch_shapes=[
            pltpu.VMEM((nb, 32), jnp.float32),
            pltpu.VMEM((nb, 1), jnp.float32),
        ],
    )(q0, q1, g2, dinv, batch_row, W2, b2, Wfc, bfc)


def kernel(x, edge_index, batch, W1, b1, W2, b2, Wfc, bfc):
    n = x.shape[0]
    e = edge_index.shape[1]
    nb = 64

    n_pad = ((n + ROWS - 1) // ROWS) * ROWS
    epw = e // NW

    src = edge_index[0]
    dst = edge_index[1]
    x_pad = jnp.pad(x, ((0, n_pad - n), (0, 0)))
    batch_row = jnp.pad(batch, (0, n_pad - n),
                        constant_values=nb).reshape(1, n_pad)

    z1 = jnp.zeros((n_pad,), jnp.float32)
    z4 = jnp.zeros((n_pad, 4), jnp.float32)
    z16 = jnp.zeros((n_pad, 16), jnp.float32)

    deg = _sc_deg(dst, z1, n_pad, epw)
    dinv, g1 = _tc_prep(deg[0].reshape(n_pad, 1),
                        deg[1].reshape(n_pad, 1), x_pad, n_pad)
    agg1 = _sc_agg(g1, src, dst, z4, n_pad, epw, 4)
    g2 = _tc_layer1(agg1[0], agg1[1], g1, dinv, W1,
                    b1.reshape(1, 16), n_pad)
    agg2 = _sc_agg(g2, src, dst, z16, n_pad, epw, 16)
    out = _tc_layer2_pool(agg2[0], agg2[1], g2, dinv, batch_row, W2,
                          b2.reshape(1, 32), Wfc, bfc.reshape(1, 1),
                          n_pad, nb)
    return out


# speedup vs baseline: 52.3190x; 52.3190x over previous
import functools

import jax
import jax.numpy as jnp
from jax import lax
from jax.experimental import pallas as pl
from jax.experimental.pallas import tpu as pltpu
from jax.experimental.pallas import tpu_sc as plsc

NC = 2
NS = 16
NW = NC * NS
CHUNK = 2000
ROWS = 2048

_mesh = plsc.VectorSubcoreMesh(core_axis_name="c", subcore_axis_name="s")


def _sc_deg(dst, zeros1, n_pad, epw):
    @functools.partial(
        pl.kernel,
        mesh=_mesh,
        out_type=jax.ShapeDtypeStruct((NC, n_pad), jnp.float32),
        scratch_types=[
            pltpu.VMEM((CHUNK,), jnp.int32),
            pltpu.VMEM((CHUNK,), jnp.float32),
            pltpu.VMEM_SHARED((n_pad,), jnp.float32),
        ],
    )
    def deg_kernel(dst_hbm, zeros_hbm, out_hbm, idx_v, ones_v, acc):
        cid = lax.axis_index("c")
        sid = lax.axis_index("s")
        wid = sid * NC + cid
        slc = n_pad // NS

        @pl.loop(0, CHUNK, step=16)
        def _(i):
            ones_v[pl.ds(i, 16)] = jnp.ones((16,), jnp.float32)

        pltpu.sync_copy(zeros_hbm.at[pl.ds(sid * slc, slc)],
                        acc.at[pl.ds(sid * slc, slc)])
        plsc.subcore_barrier()

        base = wid * epw

        @pl.loop(0, epw, step=CHUNK)
        def _(e0):
            pltpu.sync_copy(dst_hbm.at[pl.ds(base + e0, CHUNK)], idx_v)
            pltpu.sync_copy(ones_v, acc.at[idx_v], add=True)

        plsc.subcore_barrier()
        pltpu.sync_copy(acc.at[pl.ds(sid * slc, slc)],
                        out_hbm.at[cid, pl.ds(sid * slc, slc)])

    return deg_kernel(dst, zeros1)


def _sc_agg(g, src, dst, zeros, n_pad, epw, d):
    @functools.partial(
        pl.kernel,
        mesh=_mesh,
        compiler_params=pltpu.CompilerParams(use_tc_tiling_on_sc=False),
        out_type=jax.ShapeDtypeStruct((NC, n_pad, d), jnp.float32),
        scratch_types=[
            pltpu.VMEM((CHUNK,), jnp.int32),
            pltpu.VMEM((CHUNK,), jnp.int32),
            pltpu.VMEM((CHUNK, d), jnp.float32),
            pltpu.VMEM_SHARED((n_pad, d), jnp.float32),
        ],
    )
    def agg_kernel(g_hbm, src_hbm, dst_hbm, zeros_hbm, out_hbm,
                   sidx, didx, rows, acc):
        cid = lax.axis_index("c")
        sid = lax.axis_index("s")
        wid = sid * NC + cid
        slc = n_pad // NS

        pltpu.sync_copy(zeros_hbm.at[pl.ds(sid * slc, slc)],
                        acc.at[pl.ds(sid * slc, slc)])
        plsc.subcore_barrier()

        base = wid * epw

        @pl.loop(0, epw, step=CHUNK)
        def _(e0):
            pltpu.sync_copy(src_hbm.at[pl.ds(base + e0, CHUNK)], sidx)
            pltpu.sync_copy(dst_hbm.at[pl.ds(base + e0, CHUNK)], didx)
            pltpu.sync_copy(g_hbm.at[sidx], rows)
            pltpu.sync_copy(rows, acc.at[didx], add=True)

        plsc.subcore_barrier()
        pltpu.sync_copy(acc.at[pl.ds(sid * slc, slc)],
                        out_hbm.at[cid, pl.ds(sid * slc, slc)])

    return agg_kernel(g, src, dst, zeros)


def _sc_agg_fsplit(g, src, dst, zeros, n_pad, eps, d):
    @functools.partial(
        pl.kernel,
        mesh=_mesh,
        compiler_params=pltpu.CompilerParams(use_tc_tiling_on_sc=False),
        out_type=jax.ShapeDtypeStruct((NC, n_pad, d), jnp.float32),
        scratch_types=[
            pltpu.VMEM((CHUNK,), jnp.int32),
            pltpu.VMEM((CHUNK,), jnp.int32),
            pltpu.VMEM((CHUNK, d), jnp.float32),
            pltpu.VMEM_SHARED((n_pad, d), jnp.float32),
        ],
    )
    def agg_kernel(g_hbm, src_hbm, dst_hbm, zeros_hbm, out_hbm,
                   sidx, didx, rows, acc):
        cid = lax.axis_index("c")
        sid = lax.axis_index("s")
        slc = n_pad // NS

        pltpu.sync_copy(zeros_hbm.at[pl.ds(sid * slc, slc)],
                        acc.at[pl.ds(sid * slc, slc)])
        plsc.subcore_barrier()

        base = sid * eps

        @pl.loop(0, eps, step=CHUNK)
        def _(e0):
            pltpu.sync_copy(src_hbm.at[pl.ds(base + e0, CHUNK)], sidx)
            pltpu.sync_copy(dst_hbm.at[pl.ds(base + e0, CHUNK)], didx)
            pltpu.sync_copy(g_hbm.at[cid].at[sidx], rows)
            pltpu.sync_copy(rows, acc.at[didx], add=True)

        plsc.subcore_barrier()
        pltpu.sync_copy(acc.at[pl.ds(sid * slc, slc)],
                        out_hbm.at[cid, pl.ds(sid * slc, slc)])

    return agg_kernel(g, src, dst, zeros)


def _tc_prep(p0, p1, x_pad, n_pad):
    def body(p0_r, p1_r, x_r, dinv_o, g1_o):
        deg = p0_r[...] + p1_r[...] + 1.0
        dinv = lax.rsqrt(deg)
        dinv_o[...] = dinv
        g1_o[...] = x_r[...] * dinv

    grid = (n_pad // ROWS,)
    return pl.pallas_call(
        body,
        grid=grid,
        in_specs=[
            pl.BlockSpec((ROWS, 1), lambda i: (i, 0)),
            pl.BlockSpec((ROWS, 1), lambda i: (i, 0)),
            pl.BlockSpec((ROWS, 4), lambda i: (i, 0)),
        ],
        out_specs=[
            pl.BlockSpec((ROWS, 1), lambda i: (i, 0)),
            pl.BlockSpec((ROWS, 4), lambda i: (i, 0)),
        ],
        out_shape=[
            jax.ShapeDtypeStruct((n_pad, 1), jnp.float32),
            jax.ShapeDtypeStruct((n_pad, 4), jnp.float32),
        ],
    )(p0, p1, x_pad)


def _tc_layer1(a0, a1, g1, dinv, W1, b1, n_pad):
    def body(a0_r, a1_r, g1_r, dinv_r, w_r, b_r, g2_o):
        s = a0_r[...] + a1_r[...] + g1_r[...]
        h = jnp.dot(s, w_r[...], preferred_element_type=jnp.float32,
                    precision=lax.Precision.HIGHEST)
        h = h * dinv_r[...] + b_r[...]
        g2 = jnp.maximum(h, 0.0) * dinv_r[...]
        g2_o[0] = g2[:, 0:8]
        g2_o[1] = g2[:, 8:16]

    grid = (n_pad // ROWS,)
    return pl.pallas_call(
        body,
        grid=grid,
        in_specs=[
            pl.BlockSpec((ROWS, 4), lambda i: (i, 0)),
            pl.BlockSpec((ROWS, 4), lambda i: (i, 0)),
            pl.BlockSpec((ROWS, 4), lambda i: (i, 0)),
            pl.BlockSpec((ROWS, 1), lambda i: (i, 0)),
            pl.BlockSpec((4, 16), lambda i: (0, 0)),
            pl.BlockSpec((1, 16), lambda i: (0, 0)),
        ],
        out_specs=pl.BlockSpec((2, ROWS, 8), lambda i: (0, i, 0)),
        out_shape=jax.ShapeDtypeStruct((2, n_pad, 8), jnp.float32),
    )(a0, a1, g1, dinv, W1, b1)


def _tc_layer2_pool(q, g2, dinv, batch_row, W2, b2, Wfc, bfc, n_pad, nb):
    nsteps = n_pad // ROWS

    def body(q_r, g2_r, dinv_r, bt_r, w_r, b_r, wfc_r, bfc_r,
             out_o, sums, counts):
        i = pl.program_id(0)

        @pl.when(i == 0)
        def _():
            sums[...] = jnp.zeros_like(sums)
            counts[...] = jnp.zeros_like(counts)

        s = jnp.concatenate([q_r[0] + g2_r[0], q_r[1] + g2_r[1]], axis=1)
        h = jnp.dot(s, w_r[...], preferred_element_type=jnp.float32,
                    precision=lax.Precision.HIGHEST)
        h = jnp.maximum(h * dinv_r[...] + b_r[...], 0.0)
        onehot_t = (bt_r[...] ==
                    lax.broadcasted_iota(jnp.int32, (nb, 1), 0)
                    ).astype(jnp.float32)
        sums[...] += jnp.dot(onehot_t, h, preferred_element_type=jnp.float32,
                             precision=lax.Precision.HIGHEST)
        counts[...] += jnp.sum(onehot_t, axis=1, keepdims=True)

        @pl.when(i == nsteps - 1)
        def _():
            pooled = sums[...] / jnp.maximum(counts[...], 1.0)
            z = jnp.dot(pooled, wfc_r[...], preferred_element_type=jnp.float32,
                        precision=lax.Precision.HIGHEST) + bfc_r[...]
            out_o[...] = jax.nn.sigmoid(z)

    return pl.pallas_call(
        body,
        grid=(nsteps,),
        in_specs=[
            pl.BlockSpec((2, ROWS, 8), lambda i: (0, i, 0)),
            pl.BlockSpec((2, ROWS, 8), lambda i: (0, i, 0)),
            pl.BlockSpec((ROWS, 1), lambda i: (i, 0)),
            pl.BlockSpec((1, ROWS), lambda i: (0, i)),
            pl.BlockSpec((16, 32), lambda i: (0, 0)),
            pl.BlockSpec((1, 32), lambda i: (0, 0)),
            pl.BlockSpec((32, 1), lambda i: (0, 0)),
            pl.BlockSpec((1, 1), lambda i: (0, 0)),
        ],
        out_specs=pl.BlockSpec((nb, 1), lambda i: (0, 0)),
        out_shape=jax.ShapeDtypeStruct((nb, 1), jnp.float32),
        scratch_shapes=[
            pltpu.VMEM((nb, 32), jnp.float32),
            pltpu.VMEM((nb, 1), jnp.float32),
        ],
    )(q, g2, dinv, batch_row, W2, b2, Wfc, bfc)


def kernel(x, edge_index, batch, W1, b1, W2, b2, Wfc, bfc):
    n = x.shape[0]
    e = edge_index.shape[1]
    nb = 64

    n_pad = ((n + ROWS - 1) // ROWS) * ROWS
    epw = e // NW

    src = edge_index[0]
    dst = edge_index[1]
    x_pad = jnp.pad(x, ((0, n_pad - n), (0, 0)))
    batch_row = jnp.pad(batch, (0, n_pad - n),
                        constant_values=nb).reshape(1, n_pad)

    z1 = jnp.zeros((n_pad,), jnp.float32)
    z4 = jnp.zeros((n_pad, 4), jnp.float32)
    z8 = jnp.zeros((n_pad, 8), jnp.float32)

    deg = _sc_deg(dst, z1, n_pad, epw)
    dinv, g1 = _tc_prep(deg[0].reshape(n_pad, 1),
                        deg[1].reshape(n_pad, 1), x_pad, n_pad)
    agg1 = _sc_agg(g1, src, dst, z4, n_pad, epw, 4)
    g2 = _tc_layer1(agg1[0], agg1[1], g1, dinv, W1,
                    b1.reshape(1, 16), n_pad)
    agg2 = _sc_agg_fsplit(g2, src, dst, z8, n_pad, e // NS, 8)
    out = _tc_layer2_pool(agg2, g2, dinv, batch_row, W2,
                          b2.reshape(1, 32), Wfc, bfc.reshape(1, 1),
                          n_pad, nb)
    return out
